# baseline (speedup 1.0000x reference)
import functools

import jax
import jax.numpy as jnp
from jax import lax
from jax.experimental import pallas as pl
from jax.experimental.pallas import tpu as pltpu
from jax.experimental.pallas import tpu_sc as plsc

_G = 512
_BLK = 10000
_SPAN = 128

_CH = 80
_NW = 32
_NCHUNK = 100000 // _CH
_ITERS = (_NCHUNK + _NW - 1) // _NW


def _score_body(seg_ref, z_ref, w_ref, out_ref, acc_ref, m_ref, d_ref):
    i = pl.program_id(0)
    nb = pl.num_programs(0)

    @pl.when(i == 0)
    def _():
        acc_ref[...] = jnp.zeros_like(acc_ref)
        m_ref[...] = jnp.full_like(m_ref, -1e30)
        d_ref[...] = jnp.zeros_like(d_ref)

    z = z_ref[...]
    w = w_ref[...]
    s = jax.lax.dot_general(w, z, (((1,), (1,)), ((), ())),
                            preferred_element_type=jnp.float32)
    lm = jnp.max(s, axis=1, keepdims=True)
    m_old = m_ref[...]
    m_new = jnp.maximum(m_old, lm)
    scale = jnp.exp(m_old - m_new)
    e = jnp.exp(s - m_new)

    seg = seg_ref[0]

    @pl.when(lm[0, 0] > m_old[0, 0])
    def _():
        acc_ref[...] = acc_ref[...] * scale

    d_ref[...] = d_ref[...] * scale + jnp.sum(e, keepdims=True)
    m_ref[...] = m_new

    smin = jnp.min(seg)
    smax = jnp.max(seg)
    s0 = jnp.minimum((smin // 8) * 8, _G - _SPAN)
    fast = (smax - s0) < _SPAN
    zb = z.astype(jnp.bfloat16)
    eb = e.astype(jnp.bfloat16)

    @pl.when(fast)
    def _():
        hit = (jax.lax.broadcasted_iota(jnp.int32, (_SPAN, _BLK), 0)
               == (seg - s0))
        wih = hit.astype(jnp.bfloat16) * eb
        acc_ref[pl.ds(s0, _SPAN), :] += jax.lax.dot_general(
            wih, zb, (((1,), (0,)), ((), ())),
            preferred_element_type=jnp.float32)

    @pl.when(jnp.logical_not(fast))
    def _():
        hit = (jax.lax.broadcasted_iota(jnp.int32, (_G, _BLK), 0)
               == seg)
        wih = hit.astype(jnp.bfloat16) * eb
        acc_ref[...] += jax.lax.dot_general(
            wih, zb, (((1,), (0,)), ((), ())),
            preferred_element_type=jnp.float32)

    @pl.when(i == nb - 1)
    def _():
        out_ref[...] = acc_ref[...] / d_ref[...]


def _run_scores(z, seg3, w):
    n, d = z.shape
    nb = n // _BLK
    return pl.pallas_call(
        _score_body,
        grid=(nb,),
        in_specs=[
            pl.BlockSpec((1, 1, _BLK), lambda i: (i, 0, 0)),
            pl.BlockSpec((_BLK, d), lambda i: (i, 0)),
            pl.BlockSpec((1, d), lambda i: (0, 0)),
        ],
        out_specs=pl.BlockSpec((_G, d), lambda i: (0, 0)),
        out_shape=jax.ShapeDtypeStruct((_G, d), jnp.float32),
        scratch_shapes=[
            pltpu.VMEM((_G, d), jnp.float32),
            pltpu.VMEM((1, 1), jnp.float32),
            pltpu.VMEM((1, 1), jnp.float32),
        ],
    )(seg3, z, w)


@functools.partial(
    pl.kernel,
    out_type=jax.ShapeDtypeStruct((2, _G, 128), jnp.float32),
    mesh=plsc.VectorSubcoreMesh(core_axis_name="c", subcore_axis_name="s"),
    scratch_types=[
        pltpu.VMEM((_CH,), jnp.int32),
        pltpu.VMEM((_CH, 128), jnp.float32),
        pltpu.VMEM_SHARED((_G, 128), jnp.float32),
    ],
)
def _sc_counts(seg_hbm, ones_hbm, zeros_hbm, cnt_out, segbuf, onesbuf, cnt_sp):
    cid = lax.axis_index("c")
    sid = lax.axis_index("s")
    wid = sid * 2 + cid
    pltpu.sync_copy(ones_hbm, onesbuf)

    @pl.when(sid == 0)
    def _():
        pltpu.sync_copy(zeros_hbm, cnt_sp)

    plsc.subcore_barrier()

    def _chunk(it, carry):
        chunk = it * _NW + wid

        @pl.when(chunk < _NCHUNK)
        def _():
            base = chunk * _CH
            pltpu.sync_copy(seg_hbm.at[pl.ds(base, _CH)], segbuf)
            pltpu.sync_copy(onesbuf, cnt_sp.at[segbuf], add=True)

        return carry

    lax.fori_loop(0, _ITERS, _chunk, 0)
    plsc.subcore_barrier()

    rows = _G // 16
    pltpu.sync_copy(cnt_sp.at[pl.ds(sid * rows, rows)],
                    cnt_out.at[cid, pl.ds(sid * rows, rows)])


def _comb_body(acc_ref, cnt_ref, out_ref):
    c = cnt_ref[0, :, 0:1] + cnt_ref[1, :, 0:1]
    out_ref[...] = acc_ref[...] / jnp.maximum(c, 1.0)


def _combine(accd, cnt2):
    g, d = accd.shape
    return pl.pallas_call(
        _comb_body,
        in_specs=[
            pl.BlockSpec((g, d), lambda: (0, 0)),
            pl.BlockSpec((2, g, 128), lambda: (0, 0, 0)),
        ],
        out_specs=pl.BlockSpec((g, d), lambda: (0, 0)),
        out_shape=jax.ShapeDtypeStruct((g, d), jnp.float32),
    )(accd, cnt2)


@jax.jit
def _run(z, seg, seg3, w):
    accd = _run_scores(z, seg3, w)
    ones = jnp.ones((_CH, 128), jnp.float32)
    zeros = jnp.zeros((_G, 128), jnp.float32)
    cnt2 = _sc_counts(seg, ones, zeros)
    return _combine(accd, cnt2)


def kernel(z, batch_index, W, b):
    n, _ = z.shape
    seg = batch_index.astype(jnp.int32)
    seg3 = seg.reshape(n // _BLK, 1, _BLK)
    return _run(z, seg, seg3, W)


# speedup vs baseline: 2.2070x; 2.2070x over previous
import jax
import jax.numpy as jnp
from jax.experimental import pallas as pl
from jax.experimental.pallas import tpu as pltpu

_G = 512
_BLK = 10000
_SPAN = 128


def _body(seg_ref, z_ref, w_ref, out_ref, acc_ref, cnt_ref, m_ref, d_ref):
    i = pl.program_id(0)
    nb = pl.num_programs(0)

    @pl.when(i == 0)
    def _():
        acc_ref[...] = jnp.zeros_like(acc_ref)
        cnt_ref[...] = jnp.zeros_like(cnt_ref)
        m_ref[...] = jnp.full_like(m_ref, -1e30)
        d_ref[...] = jnp.zeros_like(d_ref)

    z = z_ref[...]
    w = w_ref[...]
    s = jax.lax.dot_general(w, z, (((1,), (1,)), ((), ())),
                            preferred_element_type=jnp.float32)
    lm = jnp.max(s, axis=1, keepdims=True)
    m_old = m_ref[...]
    m_new = jnp.maximum(m_old, lm)
    scale = jnp.exp(m_old - m_new)
    e = jnp.exp(s - m_new)

    seg = seg_ref[0]

    @pl.when(lm[0, 0] > m_old[0, 0])
    def _():
        acc_ref[...] = acc_ref[...] * scale

    d_ref[...] = d_ref[...] * scale + jnp.sum(e, keepdims=True)
    m_ref[...] = m_new

    smin = jnp.min(seg)
    smax = jnp.max(seg)
    s0 = jnp.minimum((smin // 8) * 8, _G - _SPAN)
    fast = (smax - s0) < _SPAN
    zb = z.astype(jnp.bfloat16)
    eb = e.astype(jnp.bfloat16)

    @pl.when(fast)
    def _():
        hit = (jax.lax.broadcasted_iota(jnp.int32, (_SPAN, _BLK), 0)
               == (seg - s0))
        wih = hit.astype(jnp.bfloat16) * eb
        acc_ref[pl.ds(s0, _SPAN), :] += jax.lax.dot_general(
            wih, zb, (((1,), (0,)), ((), ())),
            preferred_element_type=jnp.float32)
        cnt_ref[pl.ds(s0, _SPAN), :] += jnp.sum(
            hit.astype(jnp.float32), axis=1, keepdims=True)

    @pl.when(jnp.logical_not(fast))
    def _():
        hit = (jax.lax.broadcasted_iota(jnp.int32, (_G, _BLK), 0)
               == seg)
        wih = hit.astype(jnp.bfloat16) * eb
        acc_ref[...] += jax.lax.dot_general(
            wih, zb, (((1,), (0,)), ((), ())),
            preferred_element_type=jnp.float32)
        cnt_ref[...] += jnp.sum(hit.astype(jnp.float32), axis=1,
                                keepdims=True)

    @pl.when(i == nb - 1)
    def _():
        denom = d_ref[...] * jnp.maximum(cnt_ref[...], 1.0)
        out_ref[...] = acc_ref[...] / denom


@jax.jit
def _run(z, seg3, w):
    n, d = z.shape
    nb = n // _BLK
    return pl.pallas_call(
        _body,
        grid=(nb,),
        in_specs=[
            pl.BlockSpec((1, 1, _BLK), lambda i: (i, 0, 0)),
            pl.BlockSpec((_BLK, d), lambda i: (i, 0)),
            pl.BlockSpec((1, d), lambda i: (0, 0)),
        ],
        out_specs=pl.BlockSpec((_G, d), lambda i: (0, 0)),
        out_shape=jax.ShapeDtypeStruct((_G, d), jnp.float32),
        scratch_shapes=[
            pltpu.VMEM((_G, d), jnp.float32),
            pltpu.VMEM((_G, 1), jnp.float32),
            pltpu.VMEM((1, 1), jnp.float32),
            pltpu.VMEM((1, 1), jnp.float32),
        ],
    )(seg3, z, w)


def kernel(z, batch_index, W, b):
    n, _ = z.shape
    seg3 = batch_index.astype(jnp.int32).reshape(n // _BLK, 1, _BLK)
    return _run(z, seg3, W)
